# Initial kernel scaffold; baseline (speedup 1.0000x reference)
import jax
import jax.numpy as jnp
from jax.experimental import pallas as pl


def kernel(z, embedding):
    raise NotImplementedError("write your pallas kernel here")


# speedup vs baseline: 1.1115x; 1.1115x over previous
import functools

import jax
import jax.numpy as jnp
from jax import lax
from jax.experimental import pallas as pl
from jax.experimental.pallas import tpu as pltpu
from jax.experimental.pallas import tpu_sc as plsc

N_POINTS = 8192
N_CODES = 8192
DIM = 32
R_BLK = 1024
C_BLK = 2048
COMMIT = 0.25

NUM_SC_WORKERS = 32
ROWS_PER_WORKER = N_POINTS // NUM_SC_WORKERS


def _vq_body(e_ref, z_ref, en_ref, zn_ref, idx_ref, loss_ref,
             run_min, run_idx):
    j = pl.program_id(1)
    dT = lax.dot_general(
        e_ref[...], z_ref[...], (((1,), (1,)), ((), ())),
        preferred_element_type=jnp.float32)
    d = (zn_ref[0] + en_ref[...]) - 2.0 * dT
    bmin = jnp.min(d, axis=0, keepdims=True)
    rows = lax.broadcasted_iota(jnp.int32, d.shape, 0)
    bidx = jnp.min(jnp.where(d == bmin, rows, jnp.int32(2**30)),
                   axis=0, keepdims=True) + j * C_BLK

    @pl.when(j == 0)
    def _():
        run_min[...] = bmin
        run_idx[...] = bidx

    @pl.when(j > 0)
    def _():
        acc = run_min[...].astype(jnp.bfloat16).astype(jnp.float32)
        better = bmin < acc
        run_idx[...] = jnp.where(better, bidx, run_idx[...])
        run_min[...] = jnp.where(better, bmin, run_min[...])

    @pl.when(j == pl.num_programs(1) - 1)
    def _():
        idx_ref[...] = run_idx[...].reshape(1, 1, R_BLK)
        part = jnp.sum(run_min[...])

        @pl.when(pl.program_id(0) == 0)
        def _():
            loss_ref[0, 0] = part

        @pl.when(pl.program_id(0) > 0)
        def _():
            loss_ref[0, 0] += part


def _vq_argmin(e_bf, z_bf, en, zn3):
    grid = (N_POINTS // R_BLK, N_CODES // C_BLK)
    return pl.pallas_call(
        _vq_body,
        grid=grid,
        in_specs=[
            pl.BlockSpec((C_BLK, DIM), lambda i, j: (j, 0)),
            pl.BlockSpec((R_BLK, DIM), lambda i, j: (i, 0)),
            pl.BlockSpec((C_BLK, 1), lambda i, j: (j, 0)),
            pl.BlockSpec((1, 1, R_BLK), lambda i, j: (i, 0, 0)),
        ],
        out_specs=[
            pl.BlockSpec((1, 1, R_BLK), lambda i, j: (i, 0, 0)),
            pl.BlockSpec(memory_space=pltpu.SMEM, block_shape=(1, 1),
                         index_map=lambda i, j: (0, 0)),
        ],
        out_shape=[
            jax.ShapeDtypeStruct((grid[0], 1, R_BLK), jnp.int32),
            jax.ShapeDtypeStruct((1, 1), jnp.float32),
        ],
        scratch_shapes=[
            pltpu.VMEM((1, R_BLK), jnp.float32),
            pltpu.VMEM((1, R_BLK), jnp.int32),
        ],
        compiler_params=pltpu.CompilerParams(
            dimension_semantics=("arbitrary", "arbitrary")),
    )(e_bf, z_bf, en, zn3)


@functools.cache
def _sc_gather_kernel():
    mesh = plsc.VectorSubcoreMesh(core_axis_name="c", subcore_axis_name="s")

    @functools.partial(
        pl.kernel,
        mesh=mesh,
        out_type=jax.ShapeDtypeStruct((N_POINTS, DIM), jnp.float32),
        scratch_types=[
            pltpu.VMEM((ROWS_PER_WORKER,), jnp.int32),
            pltpu.VMEM((ROWS_PER_WORKER, DIM), jnp.float32),
            pltpu.SemaphoreType.DMA,
        ],
        compiler_params=pltpu.CompilerParams(use_tc_tiling_on_sc=False),
    )
    def _sc_gather(table_hbm, idx_hbm, out_hbm, idx_v, rows_v, sem):
        wid = lax.axis_index("s") * 2 + lax.axis_index("c")
        base = wid * ROWS_PER_WORKER
        pltpu.sync_copy(idx_hbm.at[pl.ds(base, ROWS_PER_WORKER)], idx_v)
        pltpu.async_copy(table_hbm.at[idx_v], rows_v, sem).wait()
        pltpu.sync_copy(rows_v, out_hbm.at[pl.ds(base, ROWS_PER_WORKER)])

    return _sc_gather


def kernel(z, embedding):
    B, D, H, W = z.shape
    z_flat = jnp.transpose(z, (0, 2, 3, 1)).reshape(-1, D)
    zn = jnp.sum(z_flat ** 2, axis=1)
    en = jnp.sum(embedding ** 2, axis=1)
    zn, en = lax.optimization_barrier((zn, en))
    z_bf = z_flat.astype(jnp.bfloat16)
    e_bf = embedding.astype(jnp.bfloat16)
    zn3 = zn.reshape(N_POINTS // R_BLK, 1, R_BLK)
    en = en.reshape(N_CODES, 1)

    idx_blocks, loss_sum = _vq_argmin(e_bf, z_bf, en, zn3)
    indices = idx_blocks.reshape(-1)

    z_q_flat = _sc_gather_kernel()(embedding, indices)

    z_q = jnp.transpose(z_q_flat.reshape(B, H, W, D), (0, 3, 1, 2))
    loss = loss_sum[0, 0] * (COMMIT / (N_POINTS * D))
    return (z_q, loss, indices.reshape(B, H, W))


# speedup vs baseline: 1.2300x; 1.1066x over previous
import functools

import jax
import jax.numpy as jnp
from jax import lax
from jax.experimental import pallas as pl
from jax.experimental.pallas import tpu as pltpu
from jax.experimental.pallas import tpu_sc as plsc

N_POINTS = 8192
N_CODES = 8192
DIM = 32
R_BLK = 2048
C_BLK = 2048
COMMIT = 0.25

NUM_SC_WORKERS = 32
ROWS_PER_WORKER = N_POINTS // NUM_SC_WORKERS


def _vq_body(e_ref, z_ref, en_ref, zn_ref, idx_ref, loss_ref,
             run_min, run_idx):
    j = pl.program_id(1)
    dT2 = lax.dot_general(
        e_ref[...], z_ref[...], (((1,), (1,)), ((), ())),
        preferred_element_type=jnp.float32)
    d = (zn_ref[0] + en_ref[...]) - dT2
    bmin = jnp.min(d, axis=0, keepdims=True)
    rows = lax.broadcasted_iota(jnp.int32, d.shape, 0)
    bidx = jnp.min(jnp.where(d == bmin, rows, jnp.int32(2**30)),
                   axis=0, keepdims=True) + j * C_BLK

    @pl.when(j == 0)
    def _():
        run_min[...] = bmin
        run_idx[...] = bidx

    @pl.when(j > 0)
    def _():
        acc = run_min[...].astype(jnp.bfloat16).astype(jnp.float32)
        better = bmin < acc
        run_idx[...] = jnp.where(better, bidx, run_idx[...])
        run_min[...] = jnp.where(better, bmin, run_min[...])

    @pl.when(j == pl.num_programs(1) - 1)
    def _():
        idx_ref[...] = run_idx[...].reshape(1, 1, R_BLK)
        part = jnp.sum(run_min[...])

        @pl.when(pl.program_id(0) == 0)
        def _():
            loss_ref[0, 0] = part

        @pl.when(pl.program_id(0) > 0)
        def _():
            loss_ref[0, 0] += part


def _vq_argmin(e_bf, z_bf, en, zn3):
    grid = (N_POINTS // R_BLK, N_CODES // C_BLK)
    return pl.pallas_call(
        _vq_body,
        grid=grid,
        in_specs=[
            pl.BlockSpec((C_BLK, DIM), lambda i, j: (j, 0)),
            pl.BlockSpec((R_BLK, DIM), lambda i, j: (i, 0)),
            pl.BlockSpec((C_BLK, 1), lambda i, j: (j, 0)),
            pl.BlockSpec((1, 1, R_BLK), lambda i, j: (i, 0, 0)),
        ],
        out_specs=[
            pl.BlockSpec((1, 1, R_BLK), lambda i, j: (i, 0, 0)),
            pl.BlockSpec(memory_space=pltpu.SMEM, block_shape=(1, 1),
                         index_map=lambda i, j: (0, 0)),
        ],
        out_shape=[
            jax.ShapeDtypeStruct((grid[0], 1, R_BLK), jnp.int32),
            jax.ShapeDtypeStruct((1, 1), jnp.float32),
        ],
        scratch_shapes=[
            pltpu.VMEM((1, R_BLK), jnp.float32),
            pltpu.VMEM((1, R_BLK), jnp.int32),
        ],
        compiler_params=pltpu.CompilerParams(
            dimension_semantics=("arbitrary", "arbitrary")),
    )(e_bf, z_bf, en, zn3)


@functools.cache
def _sc_gather_kernel():
    mesh = plsc.VectorSubcoreMesh(core_axis_name="c", subcore_axis_name="s")

    @functools.partial(
        pl.kernel,
        mesh=mesh,
        out_type=jax.ShapeDtypeStruct((N_POINTS, DIM), jnp.float32),
        scratch_types=[
            pltpu.VMEM((ROWS_PER_WORKER,), jnp.int32),
            pltpu.VMEM((ROWS_PER_WORKER, DIM), jnp.float32),
            pltpu.SemaphoreType.DMA,
        ],
        compiler_params=pltpu.CompilerParams(use_tc_tiling_on_sc=False),
    )
    def _sc_gather(table_hbm, idx_hbm, out_hbm, idx_v, rows_v, sem):
        wid = lax.axis_index("s") * 2 + lax.axis_index("c")
        base = wid * ROWS_PER_WORKER
        pltpu.sync_copy(idx_hbm.at[pl.ds(base, ROWS_PER_WORKER)], idx_v)
        pltpu.async_copy(table_hbm.at[idx_v], rows_v, sem).wait()
        pltpu.sync_copy(rows_v, out_hbm.at[pl.ds(base, ROWS_PER_WORKER)])

    return _sc_gather


def kernel(z, embedding):
    B, D, H, W = z.shape
    z_flat = jnp.transpose(z, (0, 2, 3, 1)).reshape(-1, D)
    zn = jnp.sum(z_flat ** 2, axis=1)
    en = jnp.sum(embedding ** 2, axis=1)
    zn, en = lax.optimization_barrier((zn, en))
    z_bf = z_flat.astype(jnp.bfloat16)
    e_bf = (2.0 * embedding).astype(jnp.bfloat16)
    zn3 = zn.reshape(N_POINTS // R_BLK, 1, R_BLK)
    en = en.reshape(N_CODES, 1)

    idx_blocks, loss_sum = _vq_argmin(e_bf, z_bf, en, zn3)
    indices = idx_blocks.reshape(-1)

    z_q_flat = _sc_gather_kernel()(embedding, indices)

    z_q = jnp.transpose(z_q_flat.reshape(B, H, W, D), (0, 3, 1, 2))
    loss = loss_sum[0, 0] * (COMMIT / (N_POINTS * D))
    return (z_q, loss, indices.reshape(B, H, W))


# speedup vs baseline: 1.2644x; 1.0279x over previous
import functools

import jax
import jax.numpy as jnp
from jax import lax
from jax.experimental import pallas as pl
from jax.experimental.pallas import tpu as pltpu
from jax.experimental.pallas import tpu_sc as plsc

N_POINTS = 8192
N_CODES = 8192
DIM = 32
R_BLK = 4096
C_BLK = 2048
COMMIT = 0.25

NUM_SC_WORKERS = 32
ROWS_PER_WORKER = N_POINTS // NUM_SC_WORKERS


def _vq_body(e_ref, z_ref, en_ref, zn_ref, idx_ref, loss_ref,
             run_min, run_idx):
    j = pl.program_id(1)
    dT2 = lax.dot_general(
        e_ref[...], z_ref[...], (((1,), (1,)), ((), ())),
        preferred_element_type=jnp.float32)
    d = (zn_ref[0] + en_ref[...]) - dT2
    bmin = jnp.min(d, axis=0, keepdims=True)
    rows = lax.broadcasted_iota(jnp.int32, d.shape, 0)
    bidx = jnp.min(jnp.where(d == bmin, rows, jnp.int32(2**30)),
                   axis=0, keepdims=True) + j * C_BLK

    @pl.when(j == 0)
    def _():
        run_min[...] = bmin
        run_idx[...] = bidx

    @pl.when(j > 0)
    def _():
        acc = run_min[...].astype(jnp.bfloat16).astype(jnp.float32)
        better = bmin < acc
        run_idx[...] = jnp.where(better, bidx, run_idx[...])
        run_min[...] = jnp.where(better, bmin, run_min[...])

    @pl.when(j == pl.num_programs(1) - 1)
    def _():
        idx_ref[...] = run_idx[...].reshape(1, 1, R_BLK)
        part = jnp.sum(run_min[...])

        @pl.when(pl.program_id(0) == 0)
        def _():
            loss_ref[0, 0] = part

        @pl.when(pl.program_id(0) > 0)
        def _():
            loss_ref[0, 0] += part


def _vq_argmin(e_bf, z_bf, en, zn3):
    grid = (N_POINTS // R_BLK, N_CODES // C_BLK)
    return pl.pallas_call(
        _vq_body,
        grid=grid,
        in_specs=[
            pl.BlockSpec((C_BLK, DIM), lambda i, j: (j, 0)),
            pl.BlockSpec((R_BLK, DIM), lambda i, j: (i, 0)),
            pl.BlockSpec((C_BLK, 1), lambda i, j: (j, 0)),
            pl.BlockSpec((1, 1, R_BLK), lambda i, j: (i, 0, 0)),
        ],
        out_specs=[
            pl.BlockSpec((1, 1, R_BLK), lambda i, j: (i, 0, 0)),
            pl.BlockSpec(memory_space=pltpu.SMEM, block_shape=(1, 1),
                         index_map=lambda i, j: (0, 0)),
        ],
        out_shape=[
            jax.ShapeDtypeStruct((grid[0], 1, R_BLK), jnp.int32),
            jax.ShapeDtypeStruct((1, 1), jnp.float32),
        ],
        scratch_shapes=[
            pltpu.VMEM((1, R_BLK), jnp.float32),
            pltpu.VMEM((1, R_BLK), jnp.int32),
        ],
        compiler_params=pltpu.CompilerParams(
            dimension_semantics=("arbitrary", "arbitrary")),
    )(e_bf, z_bf, en, zn3)


@functools.cache
def _sc_gather_kernel():
    mesh = plsc.VectorSubcoreMesh(core_axis_name="c", subcore_axis_name="s")

    @functools.partial(
        pl.kernel,
        mesh=mesh,
        out_type=jax.ShapeDtypeStruct((N_POINTS, DIM), jnp.float32),
        scratch_types=[
            pltpu.VMEM((ROWS_PER_WORKER,), jnp.int32),
            pltpu.VMEM((ROWS_PER_WORKER, DIM), jnp.float32),
            pltpu.SemaphoreType.DMA,
        ],
        compiler_params=pltpu.CompilerParams(use_tc_tiling_on_sc=False),
    )
    def _sc_gather(table_hbm, idx_hbm, out_hbm, idx_v, rows_v, sem):
        wid = lax.axis_index("s") * 2 + lax.axis_index("c")
        base = wid * ROWS_PER_WORKER
        pltpu.sync_copy(idx_hbm.at[pl.ds(base, ROWS_PER_WORKER)], idx_v)
        pltpu.async_copy(table_hbm.at[idx_v], rows_v, sem).wait()
        pltpu.sync_copy(rows_v, out_hbm.at[pl.ds(base, ROWS_PER_WORKER)])

    return _sc_gather


def kernel(z, embedding):
    B, D, H, W = z.shape
    z_flat = jnp.transpose(z, (0, 2, 3, 1)).reshape(-1, D)
    zn = jnp.sum(z_flat ** 2, axis=1)
    en = jnp.sum(embedding ** 2, axis=1)
    zn, en = lax.optimization_barrier((zn, en))
    z_bf = z_flat.astype(jnp.bfloat16)
    e_bf = (2.0 * embedding).astype(jnp.bfloat16)
    zn3 = zn.reshape(N_POINTS // R_BLK, 1, R_BLK)
    en = en.reshape(N_CODES, 1)

    idx_blocks, loss_sum = _vq_argmin(e_bf, z_bf, en, zn3)
    indices = idx_blocks.reshape(-1)

    z_q_flat = _sc_gather_kernel()(embedding, indices)

    z_q = jnp.transpose(z_q_flat.reshape(B, H, W, D), (0, 3, 1, 2))
    loss = loss_sum[0, 0] * (COMMIT / (N_POINTS * D))
    return (z_q, loss, indices.reshape(B, H, W))
